# Initial kernel scaffold; baseline (speedup 1.0000x reference)
import jax
import jax.numpy as jnp
from jax.experimental import pallas as pl


def kernel(x, edge_index, W1, b1, W2, b2):
    raise NotImplementedError("write your pallas kernel here")


# speedup vs baseline: 15.2710x; 15.2710x over previous
import functools

import jax
import jax.numpy as jnp
from jax import lax
from jax.experimental import pallas as pl
from jax.experimental.pallas import tpu as pltpu
from jax.experimental.pallas import tpu_sc as plsc

N = 10000
E = 320000
NFEAT = 128
NCLASS = 16
K = 10
ALPHA = 0.1

NC, NS, L = 2, 16, 16
OWN = 5120
NPAD = NC * OWN
DUMP = 1024
SROWS = OWN + DUMP
SZCH = SROWS // NS
YCH = OWN // NS
CHUNK = 128
NCHUNK = 157
EPT = NCHUNK * CHUNK
EPAD = NS * EPT

_mesh = plsc.VectorSubcoreMesh(core_axis_name="c", subcore_axis_name="s")
_sc_params = pltpu.CompilerParams(use_tc_tiling_on_sc=False)


def _f32(shape):
    return jax.ShapeDtypeStruct(shape, jnp.float32)


@functools.partial(
    pl.kernel,
    out_type=(
        _f32((NPAD, L)),
        jax.ShapeDtypeStruct((NC, NS, NCHUNK, CHUNK), jnp.int32),
    ),
    mesh=_mesh,
    scratch_types=[
        pltpu.VMEM((NCHUNK, CHUNK), jnp.int32),
        pltpu.VMEM((NCHUNK, CHUNK), jnp.int32),
        pltpu.VMEM((CHUNK, L), jnp.float32),
        pltpu.VMEM_SHARED((SROWS, L), jnp.float32),
    ],
    compiler_params=_sc_params,
)
def _deg_kernel(dst3, ones_hbm, zeros_hbm, deg_out, idx_out,
                dstv, clampv, onesv, S):
    c = lax.axis_index("c")
    s = lax.axis_index("s")
    pltpu.sync_copy(dst3.at[s], dstv)
    pltpu.sync_copy(ones_hbm, onesv)
    pltpu.sync_copy(zeros_hbm, S.at[pl.ds(s * SZCH, SZCH)])

    base = c * OWN
    lane = lax.iota(jnp.int32, L)

    def clamp_body(j, _):
        for l in range(CHUNK // L):
            dv = dstv[j, pl.ds(l * L, L)]
            local = dv - base
            ok = (local >= 0) & (local < OWN)
            spread = OWN + ((j * CHUNK + l * L + lane) & (DUMP - 1))
            clampv[j, pl.ds(l * L, L)] = jnp.where(ok, local, spread)
        return 0

    lax.fori_loop(0, NCHUNK, clamp_body, 0)
    plsc.subcore_barrier()

    def scat_body(j, _):
        pltpu.sync_copy(onesv, S.at[clampv.at[j]], add=True)
        return 0

    lax.fori_loop(0, NCHUNK, scat_body, 0)
    plsc.subcore_barrier()

    pltpu.sync_copy(S.at[pl.ds(s * YCH, YCH)],
                    deg_out.at[pl.ds(base + s * YCH, YCH)])
    pltpu.sync_copy(clampv, idx_out.at[c, s])


@functools.partial(
    pl.kernel,
    out_type=_f32((NPAD, L)),
    mesh=_mesh,
    scratch_types=[
        pltpu.VMEM((NCHUNK, CHUNK), jnp.int32),
        pltpu.VMEM((NCHUNK, CHUNK), jnp.int32),
        pltpu.VMEM((CHUNK, L), jnp.float32),
        pltpu.VMEM((YCH, L), jnp.float32),
        pltpu.VMEM((YCH, L), jnp.float32),
        pltpu.VMEM((YCH, L), jnp.float32),
        pltpu.VMEM((YCH, L), jnp.float32),
        pltpu.VMEM((YCH, L), jnp.float32),
        pltpu.VMEM_SHARED((SROWS, L), jnp.float32),
        pltpu.SemaphoreType.DMA,
    ],
    compiler_params=_sc_params,
)
def _step_kernel(y_in, src3, idx4, a_in, b_in, zeros_hbm, y_out,
                 srcv, clampv, rowv, sbuf, ybuf, abuf, bbuf, obuf, S, sem):
    c = lax.axis_index("c")
    s = lax.axis_index("s")
    pltpu.sync_copy(src3.at[s], srcv)
    pltpu.sync_copy(idx4.at[c, s], clampv)
    pltpu.sync_copy(zeros_hbm, S.at[pl.ds(s * SZCH, SZCH)])
    plsc.subcore_barrier()

    def edge_body(j, _):
        pltpu.async_copy(y_in.at[srcv.at[j]], rowv, sem).wait()
        pltpu.sync_copy(rowv, S.at[clampv.at[j]], add=True)
        return 0

    lax.fori_loop(0, NCHUNK, edge_body, 0)
    plsc.subcore_barrier()

    base = c * OWN + s * YCH
    pltpu.sync_copy(S.at[pl.ds(s * YCH, YCH)], sbuf)
    pltpu.sync_copy(y_in.at[pl.ds(base, YCH)], ybuf)
    pltpu.sync_copy(a_in.at[pl.ds(base, YCH)], abuf)
    pltpu.sync_copy(b_in.at[pl.ds(base, YCH)], bbuf)

    def upd_body(r, _):
        obuf[r, :] = abuf[r, :] * (sbuf[r, :] + ybuf[r, :]) + bbuf[r, :]
        return 0

    lax.fori_loop(0, YCH, upd_body, 0)
    pltpu.sync_copy(obuf, y_out.at[pl.ds(base, YCH)])


BLK = 1024


def _mlp_body(x_ref, w1_ref, b1_ref, w2_ref, b2_ref, deg_ref,
              y0_ref, c1_ref, c2_ref, al_ref, bl_ref):
    xb = x_ref[...]
    h1 = lax.dot_general(xb, w1_ref[...], (((1,), (1,)), ((), ())),
                         preferred_element_type=jnp.float32)
    h1 = jnp.maximum(h1 + b1_ref[...], 0.0)
    h = lax.dot_general(h1, w2_ref[...], (((1,), (1,)), ((), ())),
                        preferred_element_type=jnp.float32)
    h = h + b2_ref[...]
    deg = deg_ref[:, 0:1] + 1.0
    dinv = lax.rsqrt(deg)
    ones = jnp.ones((1, NCLASS), jnp.float32)
    y0_ref[...] = dinv * h
    c1_ref[...] = (0.9 * dinv * dinv) * ones
    c2_ref[...] = (0.1 * dinv) * h
    al_ref[...] = (0.9 * dinv) * ones
    bl_ref[...] = 0.1 * h


def _mlp_call(x_pad, W1, b1r, W2, b2r, degrows):
    grid = NPAD // BLK
    outs = [_f32((NPAD, NCLASS))] * 5
    return pl.pallas_call(
        _mlp_body,
        grid=(grid,),
        in_specs=[
            pl.BlockSpec((BLK, NFEAT), lambda i: (i, 0)),
            pl.BlockSpec((NFEAT, NFEAT), lambda i: (0, 0)),
            pl.BlockSpec((1, NFEAT), lambda i: (0, 0)),
            pl.BlockSpec((NCLASS, NFEAT), lambda i: (0, 0)),
            pl.BlockSpec((1, NCLASS), lambda i: (0, 0)),
            pl.BlockSpec((BLK, L), lambda i: (i, 0)),
        ],
        out_specs=[pl.BlockSpec((BLK, NCLASS), lambda i: (i, 0))] * 5,
        out_shape=outs,
    )(x_pad, W1, b1r, W2, b2r, degrows)


def _lsm_body(x_ref, o_ref):
    xb = x_ref[...]
    m = jnp.max(xb, axis=1, keepdims=True)
    e = jnp.exp(xb - m)
    z = jnp.sum(e, axis=1, keepdims=True)
    o_ref[...] = xb - m - jnp.log(z)


def _lsm_call(x):
    return pl.pallas_call(
        _lsm_body,
        grid=(NPAD // BLK,),
        in_specs=[pl.BlockSpec((BLK, NCLASS), lambda i: (i, 0))],
        out_specs=pl.BlockSpec((BLK, NCLASS), lambda i: (i, 0)),
        out_shape=_f32((NPAD, NCLASS)),
    )(x)


def kernel(x, edge_index, W1, b1, W2, b2):
    src = edge_index[0]
    dst = edge_index[1]
    pad = EPAD - E
    src3 = jnp.concatenate(
        [src, jnp.zeros((pad,), jnp.int32)]).reshape(NS, NCHUNK, CHUNK)
    dst3 = jnp.concatenate(
        [dst, jnp.full((pad,), -1, jnp.int32)]).reshape(NS, NCHUNK, CHUNK)
    x_pad = jnp.pad(x, ((0, NPAD - N), (0, 0)))
    ones_rows = jnp.ones((CHUNK, L), jnp.float32)
    zeros_rows = jnp.zeros((SZCH, L), jnp.float32)
    b1r = b1.reshape(1, NFEAT)
    b2r = b2.reshape(1, NCLASS)

    degrows, idx4 = _deg_kernel(dst3, ones_rows, zeros_rows)
    y0, c1, c2, al, bl = _mlp_call(x_pad, W1, b1r, W2, b2r, degrows)

    y = y0
    for _ in range(K - 1):
        y = _step_kernel(y, src3, idx4, c1, c2, zeros_rows)
    out = _step_kernel(y, src3, idx4, al, bl, zeros_rows)

    return _lsm_call(out)[:N]


# speedup vs baseline: 20.0510x; 1.3130x over previous
import functools

import jax
import jax.numpy as jnp
from jax import lax
from jax.experimental import pallas as pl
from jax.experimental.pallas import tpu as pltpu
from jax.experimental.pallas import tpu_sc as plsc

N = 10000
E = 320000
NFEAT = 128
NCLASS = 16
K = 10
ALPHA = 0.1

NC, NS, L = 2, 16, 16
OWN = 5120
NPAD = NC * OWN
DUMP = 1024
SROWS = OWN + DUMP
SZCH = SROWS // NS
YCH = OWN // NS
CHUNK = 128
NCHUNK = 160
G = 8
NG = NCHUNK // G
EPT = NCHUNK * CHUNK
EPAD = NS * EPT

_mesh = plsc.VectorSubcoreMesh(core_axis_name="c", subcore_axis_name="s")
_sc_params = pltpu.CompilerParams(use_tc_tiling_on_sc=False)


def _f32(shape):
    return jax.ShapeDtypeStruct(shape, jnp.float32)


@functools.partial(
    pl.kernel,
    out_type=(
        _f32((NPAD, L)),
        jax.ShapeDtypeStruct((NC, NS, NCHUNK, CHUNK), jnp.int32),
    ),
    mesh=_mesh,
    scratch_types=[
        pltpu.VMEM((NCHUNK, CHUNK), jnp.int32),
        pltpu.VMEM((NCHUNK, CHUNK), jnp.int32),
        pltpu.VMEM((CHUNK, L), jnp.float32),
        pltpu.VMEM_SHARED((SROWS, L), jnp.float32),
    ],
    compiler_params=_sc_params,
)
def _deg_kernel(dst3, ones_hbm, zeros_hbm, deg_out, idx_out,
                dstv, clampv, onesv, S):
    c = lax.axis_index("c")
    s = lax.axis_index("s")
    pltpu.sync_copy(dst3.at[s], dstv)
    pltpu.sync_copy(ones_hbm, onesv)
    pltpu.sync_copy(zeros_hbm, S.at[pl.ds(s * SZCH, SZCH)])

    base = c * OWN
    lane = lax.iota(jnp.int32, L)

    def clamp_body(j, _):
        for l in range(CHUNK // L):
            dv = dstv[j, pl.ds(l * L, L)]
            local = dv - base
            ok = (local >= 0) & (local < OWN)
            spread = OWN + ((j * CHUNK + l * L + lane) & (DUMP - 1))
            clampv[j, pl.ds(l * L, L)] = jnp.where(ok, local, spread)
        return 0

    lax.fori_loop(0, NCHUNK, clamp_body, 0)
    plsc.subcore_barrier()

    def scat_body(j, _):
        pltpu.sync_copy(onesv, S.at[clampv.at[j]], add=True)
        return 0

    lax.fori_loop(0, NCHUNK, scat_body, 0)
    plsc.subcore_barrier()

    pltpu.sync_copy(S.at[pl.ds(s * YCH, YCH)],
                    deg_out.at[pl.ds(base + s * YCH, YCH)])
    pltpu.sync_copy(clampv, idx_out.at[c, s])


@functools.partial(
    pl.kernel,
    out_type=_f32((NPAD, L)),
    mesh=_mesh,
    scratch_types=[
        pltpu.VMEM((NCHUNK, CHUNK), jnp.int32),
        pltpu.VMEM((NCHUNK, CHUNK), jnp.int32),
        pltpu.VMEM((2, G, CHUNK, L), jnp.float32),
        pltpu.VMEM((YCH, L), jnp.float32),
        pltpu.VMEM((YCH, L), jnp.float32),
        pltpu.VMEM((YCH, L), jnp.float32),
        pltpu.VMEM((YCH, L), jnp.float32),
        pltpu.VMEM((YCH, L), jnp.float32),
        pltpu.VMEM_SHARED((SROWS, L), jnp.float32),
        pltpu.SemaphoreType.DMA,
        pltpu.SemaphoreType.DMA,
    ],
    compiler_params=_sc_params,
)
def _step_kernel(y_in, src3, idx4, a_in, b_in, zeros_hbm, y_out,
                 srcv, clampv, rowv, sbuf, ybuf, abuf, bbuf, obuf, S,
                 gsem, ssem):
    c = lax.axis_index("c")
    s = lax.axis_index("s")
    pltpu.sync_copy(src3.at[s], srcv)
    pltpu.sync_copy(idx4.at[c, s], clampv)
    pltpu.sync_copy(zeros_hbm, S.at[pl.ds(s * SZCH, SZCH)])
    plsc.subcore_barrier()

    def fire_gather(g, half):
        for i in range(G):
            pltpu.async_copy(y_in.at[srcv.at[g * G + i]], rowv.at[half, i],
                             gsem)

    def drain_gather(half):
        for i in range(G):
            pltpu.make_async_copy(y_in.at[srcv.at[0]], rowv.at[half, i],
                                  gsem).wait()

    def fire_scatter(g, half):
        for i in range(G):
            pltpu.async_copy(rowv.at[half, i], S.at[clampv.at[g * G + i]],
                             ssem, add=True)

    def drain_scatter(half):
        for i in range(G):
            pltpu.make_async_copy(rowv.at[half, i], S.at[clampv.at[0]],
                                  ssem).wait()

    fire_gather(0, 0)

    def pair_body(p, _):
        g0 = 2 * p
        fire_gather(g0 + 1, 1)
        drain_gather(0)
        fire_scatter(g0, 0)
        drain_scatter(0)

        @pl.when(g0 + 2 < NG)
        def _():
            fire_gather(g0 + 2, 0)

        drain_gather(1)
        fire_scatter(g0 + 1, 1)
        drain_scatter(1)
        return 0

    lax.fori_loop(0, NG // 2, pair_body, 0)
    plsc.subcore_barrier()

    base = c * OWN + s * YCH
    pltpu.sync_copy(S.at[pl.ds(s * YCH, YCH)], sbuf)
    pltpu.sync_copy(y_in.at[pl.ds(base, YCH)], ybuf)
    pltpu.sync_copy(a_in.at[pl.ds(base, YCH)], abuf)
    pltpu.sync_copy(b_in.at[pl.ds(base, YCH)], bbuf)

    def upd_body(r, _):
        obuf[r, :] = abuf[r, :] * (sbuf[r, :] + ybuf[r, :]) + bbuf[r, :]
        return 0

    lax.fori_loop(0, YCH, upd_body, 0)
    pltpu.sync_copy(obuf, y_out.at[pl.ds(base, YCH)])


BLK = 1024


def _mlp_body(x_ref, w1_ref, b1_ref, w2_ref, b2_ref, deg_ref,
              y0_ref, c1_ref, c2_ref, al_ref, bl_ref):
    xb = x_ref[...]
    h1 = lax.dot_general(xb, w1_ref[...], (((1,), (1,)), ((), ())),
                         preferred_element_type=jnp.float32)
    h1 = jnp.maximum(h1 + b1_ref[...], 0.0)
    h = lax.dot_general(h1, w2_ref[...], (((1,), (1,)), ((), ())),
                        preferred_element_type=jnp.float32)
    h = h + b2_ref[...]
    deg = deg_ref[:, 0:1] + 1.0
    dinv = lax.rsqrt(deg)
    ones = jnp.ones((1, NCLASS), jnp.float32)
    y0_ref[...] = dinv * h
    c1_ref[...] = (0.9 * dinv * dinv) * ones
    c2_ref[...] = (0.1 * dinv) * h
    al_ref[...] = (0.9 * dinv) * ones
    bl_ref[...] = 0.1 * h


def _mlp_call(x_pad, W1, b1r, W2, b2r, degrows):
    grid = NPAD // BLK
    outs = [_f32((NPAD, NCLASS))] * 5
    return pl.pallas_call(
        _mlp_body,
        grid=(grid,),
        in_specs=[
            pl.BlockSpec((BLK, NFEAT), lambda i: (i, 0)),
            pl.BlockSpec((NFEAT, NFEAT), lambda i: (0, 0)),
            pl.BlockSpec((1, NFEAT), lambda i: (0, 0)),
            pl.BlockSpec((NCLASS, NFEAT), lambda i: (0, 0)),
            pl.BlockSpec((1, NCLASS), lambda i: (0, 0)),
            pl.BlockSpec((BLK, L), lambda i: (i, 0)),
        ],
        out_specs=[pl.BlockSpec((BLK, NCLASS), lambda i: (i, 0))] * 5,
        out_shape=outs,
    )(x_pad, W1, b1r, W2, b2r, degrows)


def _lsm_body(x_ref, o_ref):
    xb = x_ref[...]
    m = jnp.max(xb, axis=1, keepdims=True)
    e = jnp.exp(xb - m)
    z = jnp.sum(e, axis=1, keepdims=True)
    o_ref[...] = xb - m - jnp.log(z)


def _lsm_call(x):
    return pl.pallas_call(
        _lsm_body,
        grid=(NPAD // BLK,),
        in_specs=[pl.BlockSpec((BLK, NCLASS), lambda i: (i, 0))],
        out_specs=pl.BlockSpec((BLK, NCLASS), lambda i: (i, 0)),
        out_shape=_f32((NPAD, NCLASS)),
    )(x)


def kernel(x, edge_index, W1, b1, W2, b2):
    src = edge_index[0]
    dst = edge_index[1]
    pad = EPAD - E
    src3 = jnp.concatenate(
        [src, jnp.zeros((pad,), jnp.int32)]).reshape(NS, NCHUNK, CHUNK)
    dst3 = jnp.concatenate(
        [dst, jnp.full((pad,), -1, jnp.int32)]).reshape(NS, NCHUNK, CHUNK)
    x_pad = jnp.pad(x, ((0, NPAD - N), (0, 0)))
    ones_rows = jnp.ones((CHUNK, L), jnp.float32)
    zeros_rows = jnp.zeros((SZCH, L), jnp.float32)
    b1r = b1.reshape(1, NFEAT)
    b2r = b2.reshape(1, NCLASS)

    degrows, idx4 = _deg_kernel(dst3, ones_rows, zeros_rows)
    y0, c1, c2, al, bl = _mlp_call(x_pad, W1, b1r, W2, b2r, degrows)

    y = y0
    for _ in range(K - 1):
        y = _step_kernel(y, src3, idx4, c1, c2, zeros_rows)
    out = _step_kernel(y, src3, idx4, al, bl, zeros_rows)

    return _lsm_call(out)[:N]


# speedup vs baseline: 21.7120x; 1.0828x over previous
import functools

import jax
import jax.numpy as jnp
from jax import lax
from jax.experimental import pallas as pl
from jax.experimental.pallas import tpu as pltpu
from jax.experimental.pallas import tpu_sc as plsc

N = 10000
E = 320000
NFEAT = 128
NCLASS = 16
K = 10
ALPHA = 0.1

NC, NS, L = 2, 16, 16
OWN = 5120
NPAD = NC * OWN
DUMP = 1024
SROWS = OWN + DUMP
SZCH = SROWS // NS
YCH = OWN // NS
CHUNK = 128
NCHUNK = 160
G = 8
NG = NCHUNK // G
EPT = NCHUNK * CHUNK
EPAD = NS * EPT

_mesh = plsc.VectorSubcoreMesh(core_axis_name="c", subcore_axis_name="s")
_sc_params = pltpu.CompilerParams(use_tc_tiling_on_sc=False,
                                  needs_layout_passes=False)


def _f32(shape):
    return jax.ShapeDtypeStruct(shape, jnp.float32)


GEDGE = G * CHUNK


@functools.partial(
    pl.kernel,
    out_type=(
        _f32((NPAD, L)),
        jax.ShapeDtypeStruct((NC, NS, NCHUNK, CHUNK), jnp.int32),
        jax.ShapeDtypeStruct((NC, NS, NCHUNK, CHUNK), jnp.int32),
        jax.ShapeDtypeStruct((NC, NS, L), jnp.int32),
    ),
    mesh=_mesh,
    scratch_types=[
        pltpu.VMEM((NCHUNK, CHUNK), jnp.int32),
        pltpu.VMEM((NCHUNK, CHUNK), jnp.int32),
        pltpu.VMEM((NCHUNK, CHUNK), jnp.int32),
        pltpu.VMEM((NCHUNK, CHUNK), jnp.int32),
        pltpu.VMEM((CHUNK, L), jnp.float32),
        pltpu.VMEM((L,), jnp.int32),
        pltpu.VMEM_SHARED((SROWS, L), jnp.float32),
    ],
    compiler_params=_sc_params,
)
def _deg_kernel(src3, dst3, ones_hbm, zeros_hbm,
                deg_out, src_out, dst_out, cnt_out,
                srcv, dstv, csrc, cdst, onesv, cntv, S):
    c = lax.axis_index("c")
    s = lax.axis_index("s")
    pltpu.sync_copy(src3.at[s], srcv)
    pltpu.sync_copy(dst3.at[s], dstv)
    pltpu.sync_copy(ones_hbm, onesv)
    pltpu.sync_copy(zeros_hbm, S.at[pl.ds(s * SZCH, SZCH)])

    base = c * OWN
    lane = lax.iota(jnp.int32, L)

    def part_body(j, off):
        for l in range(CHUNK // L):
            sl = pl.ds(l * L, L)
            dv = dstv[j, sl]
            local = dv - base
            m = (local >= 0) & (local < OWN)
            mi = jnp.where(m, 1, 0)
            pos = off + plsc.cumsum(mi) - 1
            plsc.store_scatter(csrc, [pos >> 7, pos & (CHUNK - 1)],
                               srcv[j, sl], mask=m)
            plsc.store_scatter(cdst, [pos >> 7, pos & (CHUNK - 1)],
                               local, mask=m)
            off = off + lax.reduce_sum(mi, axes=(0,))
        return off

    cnt = lax.fori_loop(0, NCHUNK, part_body, jnp.int32(0))
    padded = (cnt + (GEDGE - 1)) & ~(GEDGE - 1)

    def pad_body(k, _):
        pos = cnt + k * L + lane
        m = pos < padded
        plsc.store_scatter(csrc, [pos >> 7, pos & (CHUNK - 1)],
                           jnp.zeros((L,), jnp.int32), mask=m)
        plsc.store_scatter(cdst, [pos >> 7, pos & (CHUNK - 1)],
                           OWN + (pos & (DUMP - 1)), mask=m)
        return 0

    lax.fori_loop(0, GEDGE // L, pad_body, 0)
    plsc.subcore_barrier()

    nchunks = padded >> 7

    def scat_body(j, _):
        pltpu.sync_copy(onesv, S.at[cdst.at[j]], add=True)
        return 0

    lax.fori_loop(0, nchunks, scat_body, 0)
    plsc.subcore_barrier()

    pltpu.sync_copy(S.at[pl.ds(s * YCH, YCH)],
                    deg_out.at[pl.ds(base + s * YCH, YCH)])
    pltpu.sync_copy(csrc, src_out.at[c, s])
    pltpu.sync_copy(cdst, dst_out.at[c, s])
    cntv[...] = jnp.where(lane == 0, padded >> 7, 0)
    pltpu.sync_copy(cntv, cnt_out.at[c, s])


@functools.partial(
    pl.kernel,
    out_type=_f32((NPAD, L)),
    mesh=_mesh,
    scratch_types=[
        pltpu.VMEM((NCHUNK, CHUNK), jnp.int32),
        pltpu.VMEM((NCHUNK, CHUNK), jnp.int32),
        pltpu.VMEM((L,), jnp.int32),
        pltpu.VMEM((2, G, CHUNK, L), jnp.float32),
        pltpu.VMEM((YCH, L), jnp.float32),
        pltpu.VMEM((YCH, L), jnp.float32),
        pltpu.VMEM((YCH, L), jnp.float32),
        pltpu.VMEM((YCH, L), jnp.float32),
        pltpu.VMEM((YCH, L), jnp.float32),
        pltpu.VMEM_SHARED((SROWS, L), jnp.float32),
        pltpu.SemaphoreType.DMA,
        pltpu.SemaphoreType.DMA,
    ],
    compiler_params=_sc_params,
)
def _step_kernel(y_in, src4, dst4, cnt4, a_in, b_in, zeros_hbm, y_out,
                 srcv, clampv, cntv, rowv, sbuf, ybuf, abuf, bbuf, obuf, S,
                 gsem, ssem):
    c = lax.axis_index("c")
    s = lax.axis_index("s")
    pltpu.sync_copy(src4.at[c, s], srcv)
    pltpu.sync_copy(dst4.at[c, s], clampv)
    pltpu.sync_copy(cnt4.at[c, s], cntv)
    pltpu.sync_copy(zeros_hbm, S.at[pl.ds(s * SZCH, SZCH)])
    ng = lax.reduce_sum(cntv[...], axes=(0,)) >> 3
    plsc.subcore_barrier()

    def fire_gather(g, half):
        for i in range(G):
            pltpu.async_copy(y_in.at[srcv.at[g * G + i]], rowv.at[half, i],
                             gsem)

    def drain_gather(half):
        for i in range(G):
            pltpu.make_async_copy(y_in.at[srcv.at[0]], rowv.at[half, i],
                                  gsem).wait()

    def fire_scatter(g, half):
        for i in range(G):
            pltpu.async_copy(rowv.at[half, i], S.at[clampv.at[g * G + i]],
                             ssem, add=True)

    def drain_scatter(half):
        for i in range(G):
            pltpu.make_async_copy(rowv.at[half, i], S.at[clampv.at[0]],
                                  ssem).wait()

    @pl.when(ng > 0)
    def _():
        fire_gather(0, 0)

    def stage(p, half):
        @pl.when(p + 1 < ng)
        def _():
            fire_gather(p + 1, 1 - half)

        drain_gather(half)
        fire_scatter(p, half)
        drain_scatter(half)

    def body(p, _):
        @pl.when((p & 1) == 0)
        def _():
            stage(p, 0)

        @pl.when((p & 1) == 1)
        def _():
            stage(p, 1)

        return 0

    lax.fori_loop(0, ng, body, 0)
    plsc.subcore_barrier()

    base = c * OWN + s * YCH
    pltpu.sync_copy(S.at[pl.ds(s * YCH, YCH)], sbuf)
    pltpu.sync_copy(y_in.at[pl.ds(base, YCH)], ybuf)
    pltpu.sync_copy(a_in.at[pl.ds(base, YCH)], abuf)
    pltpu.sync_copy(b_in.at[pl.ds(base, YCH)], bbuf)

    def upd_body(r, _):
        obuf[r, :] = abuf[r, :] * (sbuf[r, :] + ybuf[r, :]) + bbuf[r, :]
        return 0

    lax.fori_loop(0, YCH, upd_body, 0)
    pltpu.sync_copy(obuf, y_out.at[pl.ds(base, YCH)])


BLK = 1024


def _mlp_body(x_ref, w1_ref, b1_ref, w2_ref, b2_ref, deg_ref,
              y0_ref, c1_ref, c2_ref, al_ref, bl_ref):
    xb = x_ref[...]
    h1 = lax.dot_general(xb, w1_ref[...], (((1,), (1,)), ((), ())),
                         preferred_element_type=jnp.float32)
    h1 = jnp.maximum(h1 + b1_ref[...], 0.0)
    h = lax.dot_general(h1, w2_ref[...], (((1,), (1,)), ((), ())),
                        preferred_element_type=jnp.float32)
    h = h + b2_ref[...]
    deg = deg_ref[:, 0:1] + 1.0
    dinv = lax.rsqrt(deg)
    ones = jnp.ones((1, NCLASS), jnp.float32)
    y0_ref[...] = dinv * h
    c1_ref[...] = (0.9 * dinv * dinv) * ones
    c2_ref[...] = (0.1 * dinv) * h
    al_ref[...] = (0.9 * dinv) * ones
    bl_ref[...] = 0.1 * h


def _mlp_call(x_pad, W1, b1r, W2, b2r, degrows):
    grid = NPAD // BLK
    outs = [_f32((NPAD, NCLASS))] * 5
    return pl.pallas_call(
        _mlp_body,
        grid=(grid,),
        in_specs=[
            pl.BlockSpec((BLK, NFEAT), lambda i: (i, 0)),
            pl.BlockSpec((NFEAT, NFEAT), lambda i: (0, 0)),
            pl.BlockSpec((1, NFEAT), lambda i: (0, 0)),
            pl.BlockSpec((NCLASS, NFEAT), lambda i: (0, 0)),
            pl.BlockSpec((1, NCLASS), lambda i: (0, 0)),
            pl.BlockSpec((BLK, L), lambda i: (i, 0)),
        ],
        out_specs=[pl.BlockSpec((BLK, NCLASS), lambda i: (i, 0))] * 5,
        out_shape=outs,
    )(x_pad, W1, b1r, W2, b2r, degrows)


def _lsm_body(x_ref, o_ref):
    xb = x_ref[...]
    m = jnp.max(xb, axis=1, keepdims=True)
    e = jnp.exp(xb - m)
    z = jnp.sum(e, axis=1, keepdims=True)
    o_ref[...] = xb - m - jnp.log(z)


def _lsm_call(x):
    return pl.pallas_call(
        _lsm_body,
        grid=(NPAD // BLK,),
        in_specs=[pl.BlockSpec((BLK, NCLASS), lambda i: (i, 0))],
        out_specs=pl.BlockSpec((BLK, NCLASS), lambda i: (i, 0)),
        out_shape=_f32((NPAD, NCLASS)),
    )(x)


def kernel(x, edge_index, W1, b1, W2, b2):
    src = edge_index[0]
    dst = edge_index[1]
    pad = EPAD - E
    src3 = jnp.concatenate(
        [src, jnp.zeros((pad,), jnp.int32)]).reshape(NS, NCHUNK, CHUNK)
    dst3 = jnp.concatenate(
        [dst, jnp.full((pad,), -1, jnp.int32)]).reshape(NS, NCHUNK, CHUNK)
    x_pad = jnp.pad(x, ((0, NPAD - N), (0, 0)))
    ones_rows = jnp.ones((CHUNK, L), jnp.float32)
    zeros_rows = jnp.zeros((SZCH, L), jnp.float32)
    b1r = b1.reshape(1, NFEAT)
    b2r = b2.reshape(1, NCLASS)

    degrows, src4, dst4, cnt4 = _deg_kernel(src3, dst3, ones_rows, zeros_rows)
    y0, c1, c2, al, bl = _mlp_call(x_pad, W1, b1r, W2, b2r, degrows)

    y = y0
    for _ in range(K - 1):
        y = _step_kernel(y, src4, dst4, cnt4, c1, c2, zeros_rows)
    out = _step_kernel(y, src4, dst4, cnt4, al, bl, zeros_rows)

    return _lsm_call(out)[:N]


# speedup vs baseline: 54.0912x; 2.4913x over previous
import functools

import jax
import jax.numpy as jnp
from jax import lax
from jax.experimental import pallas as pl
from jax.experimental.pallas import tpu as pltpu
from jax.experimental.pallas import tpu_sc as plsc

N = 10000
E = 320000
NFEAT = 128
NCLASS = 16
K = 10
ALPHA = 0.1

NC, NS, L = 2, 16, 16
OWN = 5120
NPAD = NC * OWN
DUMP = 1024
SROWS = OWN + DUMP
SZCH = SROWS // NS
YCH = OWN // NS
CHUNK = 128
NCHUNK = 160
G = 8
NG = NCHUNK // G
EPT = NCHUNK * CHUNK
EPAD = NS * EPT

_mesh = plsc.VectorSubcoreMesh(core_axis_name="c", subcore_axis_name="s")
_sc_params = pltpu.CompilerParams(use_tc_tiling_on_sc=False,
                                  needs_layout_passes=False)


def _f32(shape):
    return jax.ShapeDtypeStruct(shape, jnp.float32)


GEDGE = G * CHUNK


@functools.partial(
    pl.kernel,
    out_type=(
        _f32((NPAD, L)),
        jax.ShapeDtypeStruct((NC, NS, NCHUNK, CHUNK), jnp.int32),
        jax.ShapeDtypeStruct((NC, NS, NCHUNK, CHUNK), jnp.int32),
        jax.ShapeDtypeStruct((NC, NS, L), jnp.int32),
    ),
    mesh=_mesh,
    scratch_types=[
        pltpu.VMEM((NCHUNK, CHUNK), jnp.int32),
        pltpu.VMEM((NCHUNK, CHUNK), jnp.int32),
        pltpu.VMEM((NCHUNK, CHUNK), jnp.int32),
        pltpu.VMEM((NCHUNK, CHUNK), jnp.int32),
        pltpu.VMEM((CHUNK, L), jnp.float32),
        pltpu.VMEM((L,), jnp.int32),
        pltpu.VMEM_SHARED((SROWS, L), jnp.float32),
    ],
    compiler_params=_sc_params,
)
def _deg_kernel(src3, dst3, ones_hbm, zeros_hbm,
                deg_out, src_out, dst_out, cnt_out,
                srcv, dstv, csrc, cdst, onesv, cntv, S):
    c = lax.axis_index("c")
    s = lax.axis_index("s")
    pltpu.sync_copy(src3.at[s], srcv)
    pltpu.sync_copy(dst3.at[s], dstv)
    pltpu.sync_copy(ones_hbm, onesv)
    pltpu.sync_copy(zeros_hbm, S.at[pl.ds(s * SZCH, SZCH)])

    base = c * OWN
    lane = lax.iota(jnp.int32, L)

    def part_body(j, off):
        for l in range(CHUNK // L):
            sl = pl.ds(l * L, L)
            dv = dstv[j, sl]
            local = dv - base
            m = (local >= 0) & (local < OWN)
            mi = jnp.where(m, 1, 0)
            pos = off + plsc.cumsum(mi) - 1
            plsc.store_scatter(csrc, [pos >> 7, pos & (CHUNK - 1)],
                               srcv[j, sl], mask=m)
            plsc.store_scatter(cdst, [pos >> 7, pos & (CHUNK - 1)],
                               local, mask=m)
            off = off + lax.reduce_sum(mi, axes=(0,))
        return off

    cnt = lax.fori_loop(0, NCHUNK, part_body, jnp.int32(0))
    padded = (cnt + (GEDGE - 1)) & ~(GEDGE - 1)

    def pad_body(k, _):
        pos = cnt + k * L + lane
        m = pos < padded
        plsc.store_scatter(csrc, [pos >> 7, pos & (CHUNK - 1)],
                           jnp.zeros((L,), jnp.int32), mask=m)
        plsc.store_scatter(cdst, [pos >> 7, pos & (CHUNK - 1)],
                           OWN + (pos & (DUMP - 1)), mask=m)
        return 0

    lax.fori_loop(0, GEDGE // L, pad_body, 0)
    plsc.subcore_barrier()

    nchunks = padded >> 7

    def scat_body(j, _):
        pltpu.sync_copy(onesv, S.at[cdst.at[j]], add=True)
        return 0

    lax.fori_loop(0, nchunks, scat_body, 0)
    plsc.subcore_barrier()

    pltpu.sync_copy(S.at[pl.ds(s * YCH, YCH)],
                    deg_out.at[pl.ds(base + s * YCH, YCH)])
    pltpu.sync_copy(csrc, src_out.at[c, s])
    pltpu.sync_copy(cdst, dst_out.at[c, s])
    cntv[...] = jnp.where(lane == 0, padded >> 7, 0)
    pltpu.sync_copy(cntv, cnt_out.at[c, s])


@functools.partial(
    pl.kernel,
    out_type=_f32((NPAD, L)),
    mesh=_mesh,
    scratch_types=[
        pltpu.VMEM((NCHUNK, CHUNK), jnp.int32),
        pltpu.VMEM((NCHUNK, CHUNK), jnp.int32),
        pltpu.VMEM((L,), jnp.int32),
        pltpu.VMEM((2, G, CHUNK, L), jnp.float32),
        pltpu.VMEM((YCH, L), jnp.float32),
        pltpu.VMEM((YCH, L), jnp.float32),
        pltpu.VMEM((YCH, L), jnp.float32),
        pltpu.VMEM((YCH, L), jnp.float32),
        pltpu.VMEM((YCH, L), jnp.float32),
        pltpu.VMEM_SHARED((SROWS, L), jnp.float32),
        pltpu.VMEM_SHARED((NPAD, L), jnp.float32),
        pltpu.SemaphoreType.DMA,
        pltpu.SemaphoreType.DMA,
    ],
    compiler_params=_sc_params,
)
def _step_kernel(y_in, src4, dst4, cnt4, a_in, b_in, zeros_hbm, y_out,
                 srcv, clampv, cntv, rowv, sbuf, ybuf, abuf, bbuf, obuf, S,
                 Ys, gsem, ssem):
    c = lax.axis_index("c")
    s = lax.axis_index("s")
    pltpu.sync_copy(src4.at[c, s], srcv)
    pltpu.sync_copy(dst4.at[c, s], clampv)
    pltpu.sync_copy(cnt4.at[c, s], cntv)
    pltpu.sync_copy(zeros_hbm, S.at[pl.ds(s * SZCH, SZCH)])
    pltpu.sync_copy(y_in.at[pl.ds(s * (NPAD // NS), NPAD // NS)],
                    Ys.at[pl.ds(s * (NPAD // NS), NPAD // NS)])
    ng = lax.reduce_sum(cntv[...], axes=(0,)) >> 3
    plsc.subcore_barrier()

    def fire_gather(g, half):
        for i in range(G):
            pltpu.async_copy(Ys.at[srcv.at[g * G + i]], rowv.at[half, i],
                             gsem)

    def drain_gather(half):
        for i in range(G):
            pltpu.make_async_copy(Ys.at[srcv.at[0]], rowv.at[half, i],
                                  gsem).wait()

    def fire_scatter(g, half):
        for i in range(G):
            pltpu.async_copy(rowv.at[half, i], S.at[clampv.at[g * G + i]],
                             ssem, add=True)

    def drain_scatter(half):
        for i in range(G):
            pltpu.make_async_copy(rowv.at[half, i], S.at[clampv.at[0]],
                                  ssem).wait()

    @pl.when(ng > 0)
    def _():
        fire_gather(0, 0)

    def stage(p, half):
        @pl.when(p + 1 < ng)
        def _():
            fire_gather(p + 1, 1 - half)

        drain_gather(half)
        fire_scatter(p, half)
        drain_scatter(half)

    def body(p, _):
        @pl.when((p & 1) == 0)
        def _():
            stage(p, 0)

        @pl.when((p & 1) == 1)
        def _():
            stage(p, 1)

        return 0

    lax.fori_loop(0, ng, body, 0)
    plsc.subcore_barrier()

    base = c * OWN + s * YCH
    pltpu.sync_copy(S.at[pl.ds(s * YCH, YCH)], sbuf)
    pltpu.sync_copy(y_in.at[pl.ds(base, YCH)], ybuf)
    pltpu.sync_copy(a_in.at[pl.ds(base, YCH)], abuf)
    pltpu.sync_copy(b_in.at[pl.ds(base, YCH)], bbuf)

    def upd_body(r, _):
        obuf[r, :] = abuf[r, :] * (sbuf[r, :] + ybuf[r, :]) + bbuf[r, :]
        return 0

    lax.fori_loop(0, YCH, upd_body, 0)
    pltpu.sync_copy(obuf, y_out.at[pl.ds(base, YCH)])


BLK = 1024


def _mlp_body(x_ref, w1_ref, b1_ref, w2_ref, b2_ref, deg_ref,
              y0_ref, c1_ref, c2_ref, al_ref, bl_ref):
    xb = x_ref[...]
    h1 = lax.dot_general(xb, w1_ref[...], (((1,), (1,)), ((), ())),
                         preferred_element_type=jnp.float32)
    h1 = jnp.maximum(h1 + b1_ref[...], 0.0)
    h = lax.dot_general(h1, w2_ref[...], (((1,), (1,)), ((), ())),
                        preferred_element_type=jnp.float32)
    h = h + b2_ref[...]
    deg = deg_ref[:, 0:1] + 1.0
    dinv = lax.rsqrt(deg)
    ones = jnp.ones((1, NCLASS), jnp.float32)
    y0_ref[...] = dinv * h
    c1_ref[...] = (0.9 * dinv * dinv) * ones
    c2_ref[...] = (0.1 * dinv) * h
    al_ref[...] = (0.9 * dinv) * ones
    bl_ref[...] = 0.1 * h


def _mlp_call(x_pad, W1, b1r, W2, b2r, degrows):
    grid = NPAD // BLK
    outs = [_f32((NPAD, NCLASS))] * 5
    return pl.pallas_call(
        _mlp_body,
        grid=(grid,),
        in_specs=[
            pl.BlockSpec((BLK, NFEAT), lambda i: (i, 0)),
            pl.BlockSpec((NFEAT, NFEAT), lambda i: (0, 0)),
            pl.BlockSpec((1, NFEAT), lambda i: (0, 0)),
            pl.BlockSpec((NCLASS, NFEAT), lambda i: (0, 0)),
            pl.BlockSpec((1, NCLASS), lambda i: (0, 0)),
            pl.BlockSpec((BLK, L), lambda i: (i, 0)),
        ],
        out_specs=[pl.BlockSpec((BLK, NCLASS), lambda i: (i, 0))] * 5,
        out_shape=outs,
    )(x_pad, W1, b1r, W2, b2r, degrows)


def _lsm_body(x_ref, o_ref):
    xb = x_ref[...]
    m = jnp.max(xb, axis=1, keepdims=True)
    e = jnp.exp(xb - m)
    z = jnp.sum(e, axis=1, keepdims=True)
    o_ref[...] = xb - m - jnp.log(z)


def _lsm_call(x):
    return pl.pallas_call(
        _lsm_body,
        grid=(NPAD // BLK,),
        in_specs=[pl.BlockSpec((BLK, NCLASS), lambda i: (i, 0))],
        out_specs=pl.BlockSpec((BLK, NCLASS), lambda i: (i, 0)),
        out_shape=_f32((NPAD, NCLASS)),
    )(x)


def kernel(x, edge_index, W1, b1, W2, b2):
    src = edge_index[0]
    dst = edge_index[1]
    pad = EPAD - E
    src3 = jnp.concatenate(
        [src, jnp.zeros((pad,), jnp.int32)]).reshape(NS, NCHUNK, CHUNK)
    dst3 = jnp.concatenate(
        [dst, jnp.full((pad,), -1, jnp.int32)]).reshape(NS, NCHUNK, CHUNK)
    x_pad = jnp.pad(x, ((0, NPAD - N), (0, 0)))
    ones_rows = jnp.ones((CHUNK, L), jnp.float32)
    zeros_rows = jnp.zeros((SZCH, L), jnp.float32)
    b1r = b1.reshape(1, NFEAT)
    b2r = b2.reshape(1, NCLASS)

    degrows, src4, dst4, cnt4 = _deg_kernel(src3, dst3, ones_rows, zeros_rows)
    y0, c1, c2, al, bl = _mlp_call(x_pad, W1, b1r, W2, b2r, degrows)

    y = y0
    for _ in range(K - 1):
        y = _step_kernel(y, src4, dst4, cnt4, c1, c2, zeros_rows)
    out = _step_kernel(y, src4, dst4, cnt4, al, bl, zeros_rows)

    return _lsm_call(out)[:N]


# speedup vs baseline: 54.5795x; 1.0090x over previous
import functools

import jax
import jax.numpy as jnp
from jax import lax
from jax.experimental import pallas as pl
from jax.experimental.pallas import tpu as pltpu
from jax.experimental.pallas import tpu_sc as plsc

N = 10000
E = 320000
NFEAT = 128
NCLASS = 16
K = 10
ALPHA = 0.1

NC, NS, L = 2, 16, 16
OWN = 5120
NPAD = NC * OWN
DUMP = 1024
SROWS = OWN + DUMP
SZCH = SROWS // NS
YCH = OWN // NS
CHUNK = 128
NCHUNK = 160
G = 8
NG = NCHUNK // G
EPT = NCHUNK * CHUNK
EPAD = NS * EPT

_mesh = plsc.VectorSubcoreMesh(core_axis_name="c", subcore_axis_name="s")
_sc_params = pltpu.CompilerParams(use_tc_tiling_on_sc=False,
                                  needs_layout_passes=False)


def _f32(shape):
    return jax.ShapeDtypeStruct(shape, jnp.float32)


GEDGE = G * CHUNK


@functools.partial(
    pl.kernel,
    out_type=(
        _f32((NPAD, L)),
        jax.ShapeDtypeStruct((NC, NS, NCHUNK, CHUNK), jnp.int32),
        jax.ShapeDtypeStruct((NC, NS, NCHUNK, CHUNK), jnp.int32),
        jax.ShapeDtypeStruct((NC, NS, L), jnp.int32),
    ),
    mesh=_mesh,
    scratch_types=[
        pltpu.VMEM((NCHUNK, CHUNK), jnp.int32),
        pltpu.VMEM((NCHUNK, CHUNK), jnp.int32),
        pltpu.VMEM((NCHUNK, CHUNK), jnp.int32),
        pltpu.VMEM((NCHUNK, CHUNK), jnp.int32),
        pltpu.VMEM((CHUNK, L), jnp.float32),
        pltpu.VMEM((L,), jnp.int32),
        pltpu.VMEM_SHARED((SROWS, L), jnp.float32),
        pltpu.SemaphoreType.DMA,
    ],
    compiler_params=_sc_params,
)
def _deg_kernel(src3, dst3, ones_hbm, zeros_hbm,
                deg_out, src_out, dst_out, cnt_out,
                srcv, dstv, csrc, cdst, onesv, cntv, S, ssem):
    c = lax.axis_index("c")
    s = lax.axis_index("s")
    pltpu.sync_copy(src3.at[s], srcv)
    pltpu.sync_copy(dst3.at[s], dstv)
    pltpu.sync_copy(ones_hbm, onesv)
    pltpu.sync_copy(zeros_hbm, S.at[pl.ds(s * SZCH, SZCH)])

    base = c * OWN
    lane = lax.iota(jnp.int32, L)

    def part_body(j, off):
        for l in range(CHUNK // L):
            sl = pl.ds(l * L, L)
            dv = dstv[j, sl]
            local = dv - base
            m = (local >= 0) & (local < OWN)
            mi = jnp.where(m, 1, 0)
            pos = off + plsc.cumsum(mi) - 1
            plsc.store_scatter(csrc, [pos >> 7, pos & (CHUNK - 1)],
                               srcv[j, sl], mask=m)
            plsc.store_scatter(cdst, [pos >> 7, pos & (CHUNK - 1)],
                               local, mask=m)
            off = off + lax.reduce_sum(mi, axes=(0,))
        return off

    cnt = lax.fori_loop(0, NCHUNK, part_body, jnp.int32(0))
    padded = (cnt + (GEDGE - 1)) & ~(GEDGE - 1)

    def pad_body(k, _):
        pos = cnt + k * L + lane
        m = pos < padded
        plsc.store_scatter(csrc, [pos >> 7, pos & (CHUNK - 1)],
                           jnp.zeros((L,), jnp.int32), mask=m)
        plsc.store_scatter(cdst, [pos >> 7, pos & (CHUNK - 1)],
                           OWN + (pos & (DUMP - 1)), mask=m)
        return 0

    lax.fori_loop(0, GEDGE // L, pad_body, 0)
    plsc.subcore_barrier()

    ng = padded >> 10

    def fire_ones(g):
        for i in range(G):
            pltpu.async_copy(onesv, S.at[cdst.at[g * G + i]], ssem, add=True)

    def drain_ones(g):
        for i in range(G):
            pltpu.make_async_copy(onesv, S.at[cdst.at[g * G + i]],
                                  ssem).wait()

    @pl.when(ng > 0)
    def _():
        fire_ones(0)

    def scat_body(p, _):
        @pl.when(p + 1 < ng)
        def _():
            fire_ones(p + 1)

        drain_ones(p)
        return 0

    lax.fori_loop(0, ng, scat_body, 0)
    plsc.subcore_barrier()

    pltpu.sync_copy(S.at[pl.ds(s * YCH, YCH)],
                    deg_out.at[pl.ds(base + s * YCH, YCH)])
    pltpu.sync_copy(csrc, src_out.at[c, s])
    pltpu.sync_copy(cdst, dst_out.at[c, s])
    cntv[...] = jnp.where(lane == 0, padded >> 7, 0)
    pltpu.sync_copy(cntv, cnt_out.at[c, s])


@functools.partial(
    pl.kernel,
    out_type=_f32((NPAD, L)),
    mesh=_mesh,
    scratch_types=[
        pltpu.VMEM((NCHUNK, CHUNK), jnp.int32),
        pltpu.VMEM((NCHUNK, CHUNK), jnp.int32),
        pltpu.VMEM((L,), jnp.int32),
        pltpu.VMEM((2, G, CHUNK, L), jnp.float32),
        pltpu.VMEM((YCH, L), jnp.float32),
        pltpu.VMEM((YCH, L), jnp.float32),
        pltpu.VMEM((YCH, L), jnp.float32),
        pltpu.VMEM((YCH, L), jnp.float32),
        pltpu.VMEM((YCH, L), jnp.float32),
        pltpu.VMEM_SHARED((SROWS, L), jnp.float32),
        pltpu.VMEM_SHARED((NPAD, L), jnp.float32),
        pltpu.SemaphoreType.DMA,
        pltpu.SemaphoreType.DMA,
    ],
    compiler_params=_sc_params,
)
def _step_kernel(y_in, src4, dst4, cnt4, a_in, b_in, zeros_hbm, y_out,
                 srcv, clampv, cntv, rowv, sbuf, ybuf, abuf, bbuf, obuf, S,
                 Ys, gsem, ssem):
    c = lax.axis_index("c")
    s = lax.axis_index("s")
    pltpu.sync_copy(src4.at[c, s], srcv)
    pltpu.sync_copy(dst4.at[c, s], clampv)
    pltpu.sync_copy(cnt4.at[c, s], cntv)
    pltpu.sync_copy(zeros_hbm, S.at[pl.ds(s * SZCH, SZCH)])
    pltpu.sync_copy(y_in.at[pl.ds(s * (NPAD // NS), NPAD // NS)],
                    Ys.at[pl.ds(s * (NPAD // NS), NPAD // NS)])
    ng = lax.reduce_sum(cntv[...], axes=(0,)) >> 3
    plsc.subcore_barrier()

    def fire_gather(g, half):
        for i in range(G):
            pltpu.async_copy(Ys.at[srcv.at[g * G + i]], rowv.at[half, i],
                             gsem)

    def drain_gather(half):
        for i in range(G):
            pltpu.make_async_copy(Ys.at[srcv.at[0]], rowv.at[half, i],
                                  gsem).wait()

    def fire_scatter(g, half):
        for i in range(G):
            pltpu.async_copy(rowv.at[half, i], S.at[clampv.at[g * G + i]],
                             ssem, add=True)

    def drain_scatter(half):
        for i in range(G):
            pltpu.make_async_copy(rowv.at[half, i], S.at[clampv.at[0]],
                                  ssem).wait()

    @pl.when(ng > 0)
    def _():
        fire_gather(0, 0)

    def stage(p, half):
        @pl.when(p + 1 < ng)
        def _():
            fire_gather(p + 1, 1 - half)

        drain_gather(half)
        fire_scatter(p, half)
        drain_scatter(half)

    def body(p, _):
        @pl.when((p & 1) == 0)
        def _():
            stage(p, 0)

        @pl.when((p & 1) == 1)
        def _():
            stage(p, 1)

        return 0

    lax.fori_loop(0, ng, body, 0)
    plsc.subcore_barrier()

    base = c * OWN + s * YCH
    pltpu.sync_copy(S.at[pl.ds(s * YCH, YCH)], sbuf)
    pltpu.sync_copy(y_in.at[pl.ds(base, YCH)], ybuf)
    pltpu.sync_copy(a_in.at[pl.ds(base, YCH)], abuf)
    pltpu.sync_copy(b_in.at[pl.ds(base, YCH)], bbuf)

    def upd_body(r, _):
        obuf[r, :] = abuf[r, :] * (sbuf[r, :] + ybuf[r, :]) + bbuf[r, :]
        return 0

    lax.fori_loop(0, YCH, upd_body, 0)
    pltpu.sync_copy(obuf, y_out.at[pl.ds(base, YCH)])


BLK = 1024


def _mlp_body(x_ref, w1_ref, b1_ref, w2_ref, b2_ref, deg_ref,
              y0_ref, c1_ref, c2_ref, al_ref, bl_ref):
    xb = x_ref[...]
    h1 = lax.dot_general(xb, w1_ref[...], (((1,), (1,)), ((), ())),
                         preferred_element_type=jnp.float32)
    h1 = jnp.maximum(h1 + b1_ref[...], 0.0)
    h = lax.dot_general(h1, w2_ref[...], (((1,), (1,)), ((), ())),
                        preferred_element_type=jnp.float32)
    h = h + b2_ref[...]
    deg = deg_ref[:, 0:1] + 1.0
    dinv = lax.rsqrt(deg)
    ones = jnp.ones((1, NCLASS), jnp.float32)
    y0_ref[...] = dinv * h
    c1_ref[...] = (0.9 * dinv * dinv) * ones
    c2_ref[...] = (0.1 * dinv) * h
    al_ref[...] = (0.9 * dinv) * ones
    bl_ref[...] = 0.1 * h


def _mlp_call(x_pad, W1, b1r, W2, b2r, degrows):
    grid = NPAD // BLK
    outs = [_f32((NPAD, NCLASS))] * 5
    return pl.pallas_call(
        _mlp_body,
        grid=(grid,),
        in_specs=[
            pl.BlockSpec((BLK, NFEAT), lambda i: (i, 0)),
            pl.BlockSpec((NFEAT, NFEAT), lambda i: (0, 0)),
            pl.BlockSpec((1, NFEAT), lambda i: (0, 0)),
            pl.BlockSpec((NCLASS, NFEAT), lambda i: (0, 0)),
            pl.BlockSpec((1, NCLASS), lambda i: (0, 0)),
            pl.BlockSpec((BLK, L), lambda i: (i, 0)),
        ],
        out_specs=[pl.BlockSpec((BLK, NCLASS), lambda i: (i, 0))] * 5,
        out_shape=outs,
    )(x_pad, W1, b1r, W2, b2r, degrows)


def _lsm_body(x_ref, o_ref):
    xb = x_ref[...]
    m = jnp.max(xb, axis=1, keepdims=True)
    e = jnp.exp(xb - m)
    z = jnp.sum(e, axis=1, keepdims=True)
    o_ref[...] = xb - m - jnp.log(z)


def _lsm_call(x):
    return pl.pallas_call(
        _lsm_body,
        grid=(NPAD // BLK,),
        in_specs=[pl.BlockSpec((BLK, NCLASS), lambda i: (i, 0))],
        out_specs=pl.BlockSpec((BLK, NCLASS), lambda i: (i, 0)),
        out_shape=_f32((NPAD, NCLASS)),
    )(x)


def kernel(x, edge_index, W1, b1, W2, b2):
    src = edge_index[0]
    dst = edge_index[1]
    pad = EPAD - E
    src3 = jnp.concatenate(
        [src, jnp.zeros((pad,), jnp.int32)]).reshape(NS, NCHUNK, CHUNK)
    dst3 = jnp.concatenate(
        [dst, jnp.full((pad,), -1, jnp.int32)]).reshape(NS, NCHUNK, CHUNK)
    x_pad = jnp.pad(x, ((0, NPAD - N), (0, 0)))
    ones_rows = jnp.ones((CHUNK, L), jnp.float32)
    zeros_rows = jnp.zeros((SZCH, L), jnp.float32)
    b1r = b1.reshape(1, NFEAT)
    b2r = b2.reshape(1, NCLASS)

    degrows, src4, dst4, cnt4 = _deg_kernel(src3, dst3, ones_rows, zeros_rows)
    y0, c1, c2, al, bl = _mlp_call(x_pad, W1, b1r, W2, b2r, degrows)

    y = y0
    for _ in range(K - 1):
        y = _step_kernel(y, src4, dst4, cnt4, c1, c2, zeros_rows)
    out = _step_kernel(y, src4, dst4, cnt4, al, bl, zeros_rows)

    return _lsm_call(out)[:N]


# speedup vs baseline: 63.8063x; 1.1691x over previous
import functools

import jax
import jax.numpy as jnp
from jax import lax
from jax.experimental import pallas as pl
from jax.experimental.pallas import tpu as pltpu
from jax.experimental.pallas import tpu_sc as plsc

N = 10000
E = 320000
NFEAT = 128
NCLASS = 16
K = 10
ALPHA = 0.1

NC, NS, L = 2, 16, 16
OWN = 5120
NPAD = NC * OWN
DUMP = 1024
SROWS = OWN + DUMP
SZCH = SROWS // NS
YCH = OWN // NS
CHUNK = 128
NCHUNK = 160
G = 8
NG = NCHUNK // G
EPT = NCHUNK * CHUNK
EPAD = NS * EPT

_mesh = plsc.VectorSubcoreMesh(core_axis_name="c", subcore_axis_name="s")
_sc_params = pltpu.CompilerParams(use_tc_tiling_on_sc=False,
                                  needs_layout_passes=False)


def _f32(shape):
    return jax.ShapeDtypeStruct(shape, jnp.float32)


GEDGE = G * CHUNK


@functools.partial(
    pl.kernel,
    out_type=(
        _f32((NPAD, L)),
        jax.ShapeDtypeStruct((NC, NS, NCHUNK, CHUNK), jnp.int32),
        jax.ShapeDtypeStruct((NC, NS, NCHUNK, CHUNK), jnp.int32),
        jax.ShapeDtypeStruct((NC, NS, L), jnp.int32),
    ),
    mesh=_mesh,
    scratch_types=[
        pltpu.VMEM((NCHUNK, CHUNK), jnp.int32),
        pltpu.VMEM((NCHUNK, CHUNK), jnp.int32),
        pltpu.VMEM((NCHUNK, CHUNK), jnp.int32),
        pltpu.VMEM((NCHUNK, CHUNK), jnp.int32),
        pltpu.VMEM((CHUNK, L), jnp.float32),
        pltpu.VMEM((L,), jnp.int32),
        pltpu.VMEM_SHARED((SROWS, L), jnp.float32),
        pltpu.SemaphoreType.DMA,
    ],
    compiler_params=_sc_params,
)
def _deg_kernel(src3, dst3, ones_hbm, zeros_hbm,
                deg_out, src_out, dst_out, cnt_out,
                srcv, dstv, csrc, cdst, onesv, cntv, S, ssem):
    c = lax.axis_index("c")
    s = lax.axis_index("s")
    pltpu.sync_copy(src3.at[s], srcv)
    pltpu.sync_copy(dst3.at[s], dstv)
    pltpu.sync_copy(ones_hbm, onesv)
    pltpu.sync_copy(zeros_hbm, S.at[pl.ds(s * SZCH, SZCH)])

    base = c * OWN
    lane = lax.iota(jnp.int32, L)

    def part_body(j, off):
        for l in range(CHUNK // L):
            sl = pl.ds(l * L, L)
            dv = dstv[j, sl]
            local = dv - base
            m = (local >= 0) & (local < OWN)
            mi = jnp.where(m, 1, 0)
            pos = off + plsc.cumsum(mi) - 1
            plsc.store_scatter(csrc, [pos >> 7, pos & (CHUNK - 1)],
                               srcv[j, sl], mask=m)
            plsc.store_scatter(cdst, [pos >> 7, pos & (CHUNK - 1)],
                               local, mask=m)
            off = off + lax.reduce_sum(mi, axes=(0,))
        return off

    cnt = lax.fori_loop(0, NCHUNK, part_body, jnp.int32(0))
    padded = (cnt + (GEDGE - 1)) & ~(GEDGE - 1)

    def pad_body(k, _):
        pos = cnt + k * L + lane
        m = pos < padded
        plsc.store_scatter(csrc, [pos >> 7, pos & (CHUNK - 1)],
                           jnp.zeros((L,), jnp.int32), mask=m)
        plsc.store_scatter(cdst, [pos >> 7, pos & (CHUNK - 1)],
                           OWN + (pos & (DUMP - 1)), mask=m)
        return 0

    lax.fori_loop(0, GEDGE // L, pad_body, 0)
    plsc.subcore_barrier()

    ng = padded >> 10

    def fire_ones(g):
        for i in range(G):
            pltpu.async_copy(onesv, S.at[cdst.at[g * G + i]], ssem, add=True)

    def drain_ones(g):
        for i in range(G):
            pltpu.make_async_copy(onesv, S.at[cdst.at[g * G + i]],
                                  ssem).wait()

    @pl.when(ng > 0)
    def _():
        fire_ones(0)

    def scat_body(p, _):
        @pl.when(p + 1 < ng)
        def _():
            fire_ones(p + 1)

        drain_ones(p)
        return 0

    lax.fori_loop(0, ng, scat_body, 0)
    plsc.subcore_barrier()

    pltpu.sync_copy(S.at[pl.ds(s * YCH, YCH)],
                    deg_out.at[pl.ds(base + s * YCH, YCH)])
    pltpu.sync_copy(csrc, src_out.at[c, s])
    pltpu.sync_copy(cdst, dst_out.at[c, s])
    cntv[...] = jnp.where(lane == 0, padded >> 7, 0)
    pltpu.sync_copy(cntv, cnt_out.at[c, s])


@functools.partial(
    pl.kernel,
    out_type=(_f32((NPAD, L)), _f32((NPAD, L))),
    mesh=_mesh,
    scratch_types=[
        pltpu.VMEM((NCHUNK, CHUNK), jnp.int32),
        pltpu.VMEM((NCHUNK, CHUNK), jnp.int32),
        pltpu.VMEM((L,), jnp.int32),
        pltpu.VMEM((2, G, CHUNK, L), jnp.float32),
        pltpu.VMEM((YCH, L), jnp.float32),
        pltpu.VMEM((YCH, L), jnp.float32),
        pltpu.VMEM((YCH, L), jnp.float32),
        pltpu.VMEM((YCH, L), jnp.float32),
        pltpu.VMEM((YCH, L), jnp.float32),
        pltpu.VMEM((YCH, L), jnp.float32),
        pltpu.VMEM((YCH, L), jnp.float32),
        pltpu.VMEM_SHARED((SROWS, L), jnp.float32),
        pltpu.VMEM_SHARED((NPAD, L), jnp.float32),
        pltpu.SemaphoreType.DMA,
        pltpu.SemaphoreType.DMA,
        pltpu.SemaphoreType.REGULAR,
        pltpu.SemaphoreType.REGULAR,
    ],
    compiler_params=_sc_params,
)
def _appnp_kernel(y0, src4, dst4, cnt4, a1, b1c, aL, bL, zeros_hbm,
                  out_hbm, yhbm,
                  srcv, clampv, cntv, rowv, sbuf, ya, yb,
                  abuf, bbuf, albuf, blbuf, S, Ys, gsem, ssem, xsem, asem):
    c = lax.axis_index("c")
    s = lax.axis_index("s")
    gbase = c * OWN + s * YCH
    obase = (1 - c) * OWN + s * YCH
    pltpu.sync_copy(src4.at[c, s], srcv)
    pltpu.sync_copy(dst4.at[c, s], clampv)
    pltpu.sync_copy(cnt4.at[c, s], cntv)
    pltpu.sync_copy(zeros_hbm, S.at[pl.ds(s * SZCH, SZCH)])
    pltpu.sync_copy(y0.at[pl.ds(s * (NPAD // NS), NPAD // NS)],
                    Ys.at[pl.ds(s * (NPAD // NS), NPAD // NS)])
    pltpu.sync_copy(y0.at[pl.ds(gbase, YCH)], ya)
    pltpu.sync_copy(a1.at[pl.ds(gbase, YCH)], abuf)
    pltpu.sync_copy(b1c.at[pl.ds(gbase, YCH)], bbuf)
    pltpu.sync_copy(aL.at[pl.ds(gbase, YCH)], albuf)
    pltpu.sync_copy(bL.at[pl.ds(gbase, YCH)], blbuf)
    ng = lax.reduce_sum(cntv[...], axes=(0,)) >> 3
    plsc.subcore_barrier()

    def fire_gather(g, half):
        for i in range(G):
            pltpu.async_copy(Ys.at[srcv.at[g * G + i]], rowv.at[half, i],
                             gsem)

    def drain_gather(half):
        for i in range(G):
            pltpu.make_async_copy(Ys.at[srcv.at[0]], rowv.at[half, i],
                                  gsem).wait()

    def fire_scatter(g, half):
        for i in range(G):
            pltpu.async_copy(rowv.at[half, i], S.at[clampv.at[g * G + i]],
                             ssem, add=True)

    def drain_scatter(half):
        for i in range(G):
            pltpu.make_async_copy(rowv.at[half, i], S.at[clampv.at[0]],
                                  ssem).wait()

    def stage(p, half):
        @pl.when(p + 1 < ng)
        def _():
            fire_gather(p + 1, 1 - half)

        drain_gather(half)
        fire_scatter(p, half)
        drain_scatter(half)

    def edge_body(p, _):
        @pl.when((p & 1) == 0)
        def _():
            stage(p, 0)

        @pl.when((p & 1) == 1)
        def _():
            stage(p, 1)

        return 0

    def edge_loop():
        @pl.when(ng > 0)
        def _():
            fire_gather(0, 0)

        lax.fori_loop(0, ng, edge_body, 0)

    ybufs = (ya, yb)
    for t in range(K):
        edge_loop()
        plsc.subcore_barrier()
        pltpu.sync_copy(S.at[pl.ds(s * YCH, YCH)], sbuf)
        plsc.subcore_barrier()
        ycur = ybufs[t % 2]
        ynext = ybufs[1 - t % 2]
        a_, b_ = (abuf, bbuf) if t < K - 1 else (albuf, blbuf)

        def upd_body(r, _, a_=a_, b_=b_, ycur=ycur, ynext=ynext):
            ynext[r, :] = a_[r, :] * (sbuf[r, :] + ycur[r, :]) + b_[r, :]
            return 0

        lax.fori_loop(0, YCH, upd_body, 0)
        if t == K - 1:
            pltpu.sync_copy(ynext, out_hbm.at[pl.ds(gbase, YCH)])
        else:
            pltpu.sync_copy(zeros_hbm, S.at[pl.ds(s * SZCH, SZCH)])
            pltpu.sync_copy(ynext, Ys.at[pl.ds(gbase, YCH)])
            if t > 0:
                pl.semaphore_wait(asem, 1)
            pltpu.sync_copy(ynext, yhbm.at[pl.ds(gbase, YCH)])
            plsc.subcore_barrier()
            pl.semaphore_signal(xsem, 1, core_index=1 - c)
            pl.semaphore_wait(xsem, 1)
            pltpu.sync_copy(yhbm.at[pl.ds(obase, YCH)],
                            Ys.at[pl.ds(obase, YCH)])
            plsc.subcore_barrier()
            pl.semaphore_signal(asem, 1, core_index=1 - c)


BLK = 1024


def _mlp_body(x_ref, w1_ref, b1_ref, w2_ref, b2_ref, deg_ref,
              y0_ref, c1_ref, c2_ref, al_ref, bl_ref):
    xb = x_ref[...]
    h1 = lax.dot_general(xb, w1_ref[...], (((1,), (1,)), ((), ())),
                         preferred_element_type=jnp.float32)
    h1 = jnp.maximum(h1 + b1_ref[...], 0.0)
    h = lax.dot_general(h1, w2_ref[...], (((1,), (1,)), ((), ())),
                        preferred_element_type=jnp.float32)
    h = h + b2_ref[...]
    deg = deg_ref[:, 0:1] + 1.0
    dinv = lax.rsqrt(deg)
    ones = jnp.ones((1, NCLASS), jnp.float32)
    y0_ref[...] = dinv * h
    c1_ref[...] = (0.9 * dinv * dinv) * ones
    c2_ref[...] = (0.1 * dinv) * h
    al_ref[...] = (0.9 * dinv) * ones
    bl_ref[...] = 0.1 * h


def _mlp_call(x_pad, W1, b1r, W2, b2r, degrows):
    grid = NPAD // BLK
    outs = [_f32((NPAD, NCLASS))] * 5
    return pl.pallas_call(
        _mlp_body,
        grid=(grid,),
        in_specs=[
            pl.BlockSpec((BLK, NFEAT), lambda i: (i, 0)),
            pl.BlockSpec((NFEAT, NFEAT), lambda i: (0, 0)),
            pl.BlockSpec((1, NFEAT), lambda i: (0, 0)),
            pl.BlockSpec((NCLASS, NFEAT), lambda i: (0, 0)),
            pl.BlockSpec((1, NCLASS), lambda i: (0, 0)),
            pl.BlockSpec((BLK, L), lambda i: (i, 0)),
        ],
        out_specs=[pl.BlockSpec((BLK, NCLASS), lambda i: (i, 0))] * 5,
        out_shape=outs,
    )(x_pad, W1, b1r, W2, b2r, degrows)


def _lsm_body(x_ref, o_ref):
    xb = x_ref[...]
    m = jnp.max(xb, axis=1, keepdims=True)
    e = jnp.exp(xb - m)
    z = jnp.sum(e, axis=1, keepdims=True)
    o_ref[...] = xb - m - jnp.log(z)


def _lsm_call(x):
    return pl.pallas_call(
        _lsm_body,
        grid=(NPAD // BLK,),
        in_specs=[pl.BlockSpec((BLK, NCLASS), lambda i: (i, 0))],
        out_specs=pl.BlockSpec((BLK, NCLASS), lambda i: (i, 0)),
        out_shape=_f32((NPAD, NCLASS)),
    )(x)


def kernel(x, edge_index, W1, b1, W2, b2):
    src = edge_index[0]
    dst = edge_index[1]
    pad = EPAD - E
    src3 = jnp.concatenate(
        [src, jnp.zeros((pad,), jnp.int32)]).reshape(NS, NCHUNK, CHUNK)
    dst3 = jnp.concatenate(
        [dst, jnp.full((pad,), -1, jnp.int32)]).reshape(NS, NCHUNK, CHUNK)
    x_pad = jnp.pad(x, ((0, NPAD - N), (0, 0)))
    ones_rows = jnp.ones((CHUNK, L), jnp.float32)
    zeros_rows = jnp.zeros((SZCH, L), jnp.float32)
    b1r = b1.reshape(1, NFEAT)
    b2r = b2.reshape(1, NCLASS)

    degrows, src4, dst4, cnt4 = _deg_kernel(src3, dst3, ones_rows, zeros_rows)
    y0, c1, c2, al, bl = _mlp_call(x_pad, W1, b1r, W2, b2r, degrows)

    out, _ = _appnp_kernel(y0, src4, dst4, cnt4, c1, c2, al, bl, zeros_rows)

    return _lsm_call(out)[:N]


# speedup vs baseline: 63.8103x; 1.0001x over previous
import functools

import jax
import jax.numpy as jnp
from jax import lax
from jax.experimental import pallas as pl
from jax.experimental.pallas import tpu as pltpu
from jax.experimental.pallas import tpu_sc as plsc

N = 10000
E = 320000
NFEAT = 128
NCLASS = 16
K = 10
ALPHA = 0.1

NC, NS, L = 2, 16, 16
OWN = 5120
NPAD = NC * OWN
DUMP = 1024
SROWS = OWN + DUMP
SZCH = SROWS // NS
YCH = OWN // NS
CHUNK = 128
NCHUNK = 160
G = 8
NG = NCHUNK // G
EPT = NCHUNK * CHUNK
EPAD = NS * EPT

_sc_params = pltpu.CompilerParams(use_tc_tiling_on_sc=False,
                                  needs_layout_passes=False)


def _f32(shape):
    return jax.ShapeDtypeStruct(shape, jnp.float32)


GEDGE = G * CHUNK


_DEG_OUT = (
    _f32((NPAD, L)),
    jax.ShapeDtypeStruct((NC, NS, NCHUNK, CHUNK), jnp.int32),
    jax.ShapeDtypeStruct((NC, NS, NCHUNK, CHUNK), jnp.int32),
    jax.ShapeDtypeStruct((NC, NS, L), jnp.int32),
)
_DEG_SCRATCH = [
    pltpu.VMEM((NCHUNK, CHUNK), jnp.int32),
    pltpu.VMEM((NCHUNK, CHUNK), jnp.int32),
    pltpu.VMEM((NCHUNK, CHUNK), jnp.int32),
    pltpu.VMEM((NCHUNK, CHUNK), jnp.int32),
    pltpu.VMEM((CHUNK, L), jnp.float32),
    pltpu.VMEM((L,), jnp.int32),
    pltpu.VMEM_SHARED((SROWS, L), jnp.float32),
    pltpu.SemaphoreType.DMA,
]


def _deg_body(src3, dst3, ones_hbm, zeros_hbm,
                deg_out, src_out, dst_out, cnt_out,
                srcv, dstv, csrc, cdst, onesv, cntv, S, ssem):
    c = lax.axis_index("c")
    s = lax.axis_index("s")
    pltpu.sync_copy(src3.at[s], srcv)
    pltpu.sync_copy(dst3.at[s], dstv)
    pltpu.sync_copy(ones_hbm, onesv)
    pltpu.sync_copy(zeros_hbm, S.at[pl.ds(s * SZCH, SZCH)])

    base = c * OWN
    lane = lax.iota(jnp.int32, L)

    def part_body(j, off):
        for l in range(CHUNK // L):
            sl = pl.ds(l * L, L)
            dv = dstv[j, sl]
            local = dv - base
            m = (local >= 0) & (local < OWN)
            mi = jnp.where(m, 1, 0)
            pos = off + plsc.cumsum(mi) - 1
            plsc.store_scatter(csrc, [pos >> 7, pos & (CHUNK - 1)],
                               srcv[j, sl], mask=m)
            plsc.store_scatter(cdst, [pos >> 7, pos & (CHUNK - 1)],
                               local, mask=m)
            off = off + lax.reduce_sum(mi, axes=(0,))
        return off

    cnt = lax.fori_loop(0, NCHUNK, part_body, jnp.int32(0))
    padded = (cnt + (GEDGE - 1)) & ~(GEDGE - 1)

    def pad_body(k, _):
        pos = cnt + k * L + lane
        m = pos < padded
        plsc.store_scatter(csrc, [pos >> 7, pos & (CHUNK - 1)],
                           jnp.zeros((L,), jnp.int32), mask=m)
        plsc.store_scatter(cdst, [pos >> 7, pos & (CHUNK - 1)],
                           OWN + (pos & (DUMP - 1)), mask=m)
        return 0

    lax.fori_loop(0, GEDGE // L, pad_body, 0)
    plsc.subcore_barrier()

    ng = padded >> 10

    def fire_ones(g):
        for i in range(G):
            pltpu.async_copy(onesv, S.at[cdst.at[g * G + i]], ssem, add=True)

    def drain_ones(g):
        for i in range(G):
            pltpu.make_async_copy(onesv, S.at[cdst.at[g * G + i]],
                                  ssem).wait()

    @pl.when(ng > 0)
    def _():
        fire_ones(0)

    def scat_body(p, _):
        @pl.when(p + 1 < ng)
        def _():
            fire_ones(p + 1)

        drain_ones(p)
        return 0

    lax.fori_loop(0, ng, scat_body, 0)
    plsc.subcore_barrier()

    pltpu.sync_copy(S.at[pl.ds(s * YCH, YCH)],
                    deg_out.at[pl.ds(base + s * YCH, YCH)])
    pltpu.sync_copy(csrc, src_out.at[c, s])
    pltpu.sync_copy(cdst, dst_out.at[c, s])
    cntv[...] = jnp.where(lane == 0, padded >> 7, 0)
    pltpu.sync_copy(cntv, cnt_out.at[c, s])


_PROP_OUT = (_f32((NPAD, L)), _f32((NPAD, L)))
_PROP_SCRATCH = [
    pltpu.VMEM((NCHUNK, CHUNK), jnp.int32),
    pltpu.VMEM((NCHUNK, CHUNK), jnp.int32),
    pltpu.VMEM((L,), jnp.int32),
    pltpu.VMEM((2, G, CHUNK, L), jnp.float32),
    pltpu.VMEM((YCH, L), jnp.float32),
    pltpu.VMEM((YCH, L), jnp.float32),
    pltpu.VMEM((YCH, L), jnp.float32),
    pltpu.VMEM((YCH, L), jnp.float32),
    pltpu.VMEM((YCH, L), jnp.float32),
    pltpu.VMEM((YCH, L), jnp.float32),
    pltpu.VMEM((YCH, L), jnp.float32),
    pltpu.VMEM_SHARED((SROWS, L), jnp.float32),
    pltpu.VMEM_SHARED((NPAD, L), jnp.float32),
    pltpu.SemaphoreType.DMA,
    pltpu.SemaphoreType.DMA,
    pltpu.SemaphoreType.REGULAR,
    pltpu.SemaphoreType.REGULAR,
]


def _appnp_body(y0, src4, dst4, cnt4, a1, b1c, aL, bL, zeros_hbm,
                  out_hbm, yhbm,
                  srcv, clampv, cntv, rowv, sbuf, ya, yb,
                  abuf, bbuf, albuf, blbuf, S, Ys, gsem, ssem, xsem, asem):
    c = lax.axis_index("c")
    s = lax.axis_index("s")
    gbase = c * OWN + s * YCH
    obase = (1 - c) * OWN + s * YCH
    pltpu.sync_copy(src4.at[c, s], srcv)
    pltpu.sync_copy(dst4.at[c, s], clampv)
    pltpu.sync_copy(cnt4.at[c, s], cntv)
    pltpu.sync_copy(zeros_hbm, S.at[pl.ds(s * SZCH, SZCH)])
    pltpu.sync_copy(y0.at[pl.ds(s * (NPAD // NS), NPAD // NS)],
                    Ys.at[pl.ds(s * (NPAD // NS), NPAD // NS)])
    pltpu.sync_copy(y0.at[pl.ds(gbase, YCH)], ya)
    pltpu.sync_copy(a1.at[pl.ds(gbase, YCH)], abuf)
    pltpu.sync_copy(b1c.at[pl.ds(gbase, YCH)], bbuf)
    pltpu.sync_copy(aL.at[pl.ds(gbase, YCH)], albuf)
    pltpu.sync_copy(bL.at[pl.ds(gbase, YCH)], blbuf)
    ng = lax.reduce_sum(cntv[...], axes=(0,)) >> 3
    plsc.subcore_barrier()

    def fire_gather(g, half):
        for i in range(G):
            pltpu.async_copy(Ys.at[srcv.at[g * G + i]], rowv.at[half, i],
                             gsem)

    def drain_gather(half):
        for i in range(G):
            pltpu.make_async_copy(Ys.at[srcv.at[0]], rowv.at[half, i],
                                  gsem).wait()

    def fire_scatter(g, half):
        for i in range(G):
            pltpu.async_copy(rowv.at[half, i], S.at[clampv.at[g * G + i]],
                             ssem, add=True)

    def drain_scatter(half):
        for i in range(G):
            pltpu.make_async_copy(rowv.at[half, i], S.at[clampv.at[0]],
                                  ssem).wait()

    def stage(p, half):
        @pl.when(p + 1 < ng)
        def _():
            fire_gather(p + 1, 1 - half)

        drain_gather(half)
        fire_scatter(p, half)
        drain_scatter(half)

    def edge_body(p, _):
        @pl.when((p & 1) == 0)
        def _():
            stage(p, 0)

        @pl.when((p & 1) == 1)
        def _():
            stage(p, 1)

        return 0

    def edge_loop():
        @pl.when(ng > 0)
        def _():
            fire_gather(0, 0)

        lax.fori_loop(0, ng, edge_body, 0)

    ybufs = (ya, yb)
    for t in range(K):
        edge_loop()
        plsc.subcore_barrier()
        pltpu.sync_copy(S.at[pl.ds(s * YCH, YCH)], sbuf)
        plsc.subcore_barrier()
        ycur = ybufs[t % 2]
        ynext = ybufs[1 - t % 2]
        a_, b_ = (abuf, bbuf) if t < K - 1 else (albuf, blbuf)

        def upd_body(r, _, a_=a_, b_=b_, ycur=ycur, ynext=ynext):
            ynext[r, :] = a_[r, :] * (sbuf[r, :] + ycur[r, :]) + b_[r, :]
            return 0

        lax.fori_loop(0, YCH, upd_body, 0)
        if t == K - 1:
            pltpu.sync_copy(ynext, out_hbm.at[pl.ds(gbase, YCH)])
        else:
            pltpu.sync_copy(zeros_hbm, S.at[pl.ds(s * SZCH, SZCH)])
            pltpu.sync_copy(ynext, Ys.at[pl.ds(gbase, YCH)])
            if t > 0:
                pl.semaphore_wait(asem, 1)
            pltpu.sync_copy(ynext, yhbm.at[pl.ds(gbase, YCH)])
            plsc.subcore_barrier()
            pl.semaphore_signal(xsem, 1, core_index=1 - c)
            pl.semaphore_wait(xsem, 1)
            pltpu.sync_copy(yhbm.at[pl.ds(obase, YCH)],
                            Ys.at[pl.ds(obase, YCH)])
            plsc.subcore_barrier()
            pl.semaphore_signal(asem, 1, core_index=1 - c)


@functools.cache
def _sc_kernels():
    mesh = plsc.VectorSubcoreMesh(core_axis_name="c", subcore_axis_name="s",
                                  num_cores=NC, num_subcores=NS)
    deg = pl.kernel(_deg_body, out_type=_DEG_OUT, mesh=mesh,
                    scratch_types=_DEG_SCRATCH, compiler_params=_sc_params)
    prop = pl.kernel(_appnp_body, out_type=_PROP_OUT, mesh=mesh,
                     scratch_types=_PROP_SCRATCH, compiler_params=_sc_params)
    return deg, prop


BLK = 1024


def _mlp_body(x_ref, w1_ref, b1_ref, w2_ref, b2_ref, deg_ref,
              y0_ref, c1_ref, c2_ref, al_ref, bl_ref):
    xb = x_ref[...]
    h1 = lax.dot_general(xb, w1_ref[...], (((1,), (1,)), ((), ())),
                         preferred_element_type=jnp.float32)
    h1 = jnp.maximum(h1 + b1_ref[...], 0.0)
    h = lax.dot_general(h1, w2_ref[...], (((1,), (1,)), ((), ())),
                        preferred_element_type=jnp.float32)
    h = h + b2_ref[...]
    deg = deg_ref[:, 0:1] + 1.0
    dinv = lax.rsqrt(deg)
    ones = jnp.ones((1, NCLASS), jnp.float32)
    y0_ref[...] = dinv * h
    c1_ref[...] = (0.9 * dinv * dinv) * ones
    c2_ref[...] = (0.1 * dinv) * h
    al_ref[...] = (0.9 * dinv) * ones
    bl_ref[...] = 0.1 * h


def _mlp_call(x_pad, W1, b1r, W2, b2r, degrows):
    grid = NPAD // BLK
    outs = [_f32((NPAD, NCLASS))] * 5
    return pl.pallas_call(
        _mlp_body,
        grid=(grid,),
        in_specs=[
            pl.BlockSpec((BLK, NFEAT), lambda i: (i, 0)),
            pl.BlockSpec((NFEAT, NFEAT), lambda i: (0, 0)),
            pl.BlockSpec((1, NFEAT), lambda i: (0, 0)),
            pl.BlockSpec((NCLASS, NFEAT), lambda i: (0, 0)),
            pl.BlockSpec((1, NCLASS), lambda i: (0, 0)),
            pl.BlockSpec((BLK, L), lambda i: (i, 0)),
        ],
        out_specs=[pl.BlockSpec((BLK, NCLASS), lambda i: (i, 0))] * 5,
        out_shape=outs,
    )(x_pad, W1, b1r, W2, b2r, degrows)


def _lsm_body(x_ref, o_ref):
    xb = x_ref[...]
    m = jnp.max(xb, axis=1, keepdims=True)
    e = jnp.exp(xb - m)
    z = jnp.sum(e, axis=1, keepdims=True)
    o_ref[...] = xb - m - jnp.log(z)


def _lsm_call(x):
    return pl.pallas_call(
        _lsm_body,
        grid=(NPAD // BLK,),
        in_specs=[pl.BlockSpec((BLK, NCLASS), lambda i: (i, 0))],
        out_specs=pl.BlockSpec((BLK, NCLASS), lambda i: (i, 0)),
        out_shape=_f32((NPAD, NCLASS)),
    )(x)


def kernel(x, edge_index, W1, b1, W2, b2):
    src = edge_index[0]
    dst = edge_index[1]
    pad = EPAD - E
    src3 = jnp.concatenate(
        [src, jnp.zeros((pad,), jnp.int32)]).reshape(NS, NCHUNK, CHUNK)
    dst3 = jnp.concatenate(
        [dst, jnp.full((pad,), -1, jnp.int32)]).reshape(NS, NCHUNK, CHUNK)
    x_pad = jnp.pad(x, ((0, NPAD - N), (0, 0)))
    ones_rows = jnp.ones((CHUNK, L), jnp.float32)
    zeros_rows = jnp.zeros((SZCH, L), jnp.float32)
    b1r = b1.reshape(1, NFEAT)
    b2r = b2.reshape(1, NCLASS)

    deg_kernel, appnp_kernel = _sc_kernels()
    degrows, src4, dst4, cnt4 = deg_kernel(src3, dst3, ones_rows, zeros_rows)
    y0, c1, c2, al, bl = _mlp_call(x_pad, W1, b1r, W2, b2r, degrows)

    out, _ = appnp_kernel(y0, src4, dst4, cnt4, c1, c2, al, bl, zeros_rows)

    return _lsm_call(out)[:N]
